# Initial kernel scaffold; baseline (speedup 1.0000x reference)
import jax
import jax.numpy as jnp
from jax.experimental import pallas as pl


def kernel(idx, targets, token_emb_table):
    raise NotImplementedError("write your pallas kernel here")


# speedup vs baseline: 1.4137x; 1.4137x over previous
import functools

import jax
import jax.numpy as jnp
from jax import lax
from jax.experimental import pallas as pl
from jax.experimental.pallas import tpu as pltpu
from jax.experimental.pallas import tpu_sc as plsc

_V = 1000
_N = 204800
_NC, _NS = 2, 16
_NW = _NC * _NS
_PER_W = _N // _NW
_CHUNK = 32
_NCHUNK = _PER_W // _CHUNK


def _lse_body(t_ref, o_ref):
    t = t_ref[...]
    m = jnp.max(t, axis=1, keepdims=True)
    o_ref[...] = m + jnp.log(jnp.sum(jnp.exp(t - m), axis=1, keepdims=True))


_lse_call = pl.pallas_call(
    _lse_body,
    out_shape=jax.ShapeDtypeStruct((_V, 1), jnp.float32),
)


def _loss_body(p_ref, o_ref):
    o_ref[0, 0] = jnp.sum(p_ref[...]) * (1.0 / _N)


_loss_call = pl.pallas_call(
    _loss_body,
    out_shape=jax.ShapeDtypeStruct((1, 1), jnp.float32),
    out_specs=pl.BlockSpec(memory_space=pltpu.SMEM),
)


_mesh = plsc.VectorSubcoreMesh(core_axis_name="c", subcore_axis_name="s")


@functools.partial(
    pl.kernel,
    out_type=(
        jax.ShapeDtypeStruct((_N, _V), jnp.float32),
        jax.ShapeDtypeStruct((_NW, 16), jnp.float32),
    ),
    mesh=_mesh,
    compiler_params=pltpu.CompilerParams(
        use_tc_tiling_on_sc=False, needs_layout_passes=False),
    scratch_types=[
        pltpu.VMEM((_NCHUNK, _CHUNK), jnp.int32),
        pltpu.VMEM((_NCHUNK, _CHUNK), jnp.int32),
        pltpu.VMEM((_V,), jnp.float32),
        pltpu.VMEM((_CHUNK, _V), jnp.float32),
        pltpu.VMEM((16,), jnp.float32),
        pltpu.SemaphoreType.DMA,
    ],
)
def _sc_gather(table_hbm, idx_hbm, tgt_hbm, lse_hbm, out_hbm, part_hbm,
               idx_v, tgt_v, lse_v, buf, acc_v, sem):
    wid = lax.axis_index("s") * _NC + lax.axis_index("c")
    base = wid * _PER_W
    pltpu.sync_copy(idx_hbm.at[wid], idx_v)
    pltpu.sync_copy(tgt_hbm.at[wid], tgt_v)
    pltpu.sync_copy(lse_hbm, lse_v)
    acc_v[...] = jnp.zeros((16,), jnp.float32)

    def chunk_body(j, carry):
        pltpu.async_copy(table_hbm.at[idx_v.at[j]], buf, sem).wait()
        for b in range(_CHUNK // 16):
            rid = lax.iota(jnp.int32, 16) + (b * 16)
            tg = tgt_v[j, pl.ds(b * 16, 16)]
            ii = idx_v[j, pl.ds(b * 16, 16)]
            picked = plsc.load_gather(buf, [rid, tg])
            lseg = plsc.load_gather(lse_v, [ii])
            acc_v[...] = acc_v[...] + (lseg - picked)
        pltpu.sync_copy(buf, out_hbm.at[pl.ds(base + j * _CHUNK, _CHUNK)])
        return carry

    lax.fori_loop(0, _NCHUNK, chunk_body, 0)
    pltpu.sync_copy(acc_v, part_hbm.at[wid])


def kernel(idx, targets, token_emb_table):
    idx_r = idx.reshape(_NW, _NCHUNK, _CHUNK)
    tgt_r = targets.reshape(_NW, _NCHUNK, _CHUNK)
    lse = _lse_call(token_emb_table).reshape(_V)
    logits2, partials = _sc_gather(token_emb_table, idx_r, tgt_r, lse)
    loss = _loss_call(partials)[0, 0]
    return (logits2, loss)


# speedup vs baseline: 1.4965x; 1.0586x over previous
import functools

import jax
import jax.numpy as jnp
from jax import lax
from jax.experimental import pallas as pl
from jax.experimental.pallas import tpu as pltpu
from jax.experimental.pallas import tpu_sc as plsc

_V = 1000
_N = 204800
_NC, _NS = 2, 16
_NW = _NC * _NS
_PER_W = _N // _NW
_CHUNK = 32
_NCHUNK = _PER_W // _CHUNK


def _lse_body(t_ref, o_ref):
    t = t_ref[...]
    m = jnp.max(t, axis=1, keepdims=True)
    o_ref[...] = m + jnp.log(jnp.sum(jnp.exp(t - m), axis=1, keepdims=True))


_lse_call = pl.pallas_call(
    _lse_body,
    out_shape=jax.ShapeDtypeStruct((_V, 1), jnp.float32),
)


def _loss_body(p_ref, o_ref):
    o_ref[0, 0] = jnp.sum(p_ref[...]) * (1.0 / _N)


_loss_call = pl.pallas_call(
    _loss_body,
    out_shape=jax.ShapeDtypeStruct((1, 1), jnp.float32),
    out_specs=pl.BlockSpec(memory_space=pltpu.SMEM),
)


_mesh = plsc.VectorSubcoreMesh(core_axis_name="c", subcore_axis_name="s")


@functools.partial(
    pl.kernel,
    out_type=(
        jax.ShapeDtypeStruct((_N, _V), jnp.float32),
        jax.ShapeDtypeStruct((_NW, 16), jnp.float32),
    ),
    mesh=_mesh,
    compiler_params=pltpu.CompilerParams(
        use_tc_tiling_on_sc=False, needs_layout_passes=False),
    scratch_types=[
        pltpu.VMEM((_NCHUNK, _CHUNK), jnp.int32),
        pltpu.VMEM((_NCHUNK, _CHUNK), jnp.int32),
        pltpu.VMEM((_V,), jnp.float32),
        pltpu.VMEM((_CHUNK, _V), jnp.float32),
        pltpu.VMEM((_CHUNK, _V), jnp.float32),
        pltpu.VMEM((16,), jnp.float32),
        pltpu.SemaphoreType.DMA,
        pltpu.SemaphoreType.DMA,
    ],
)
def _sc_gather(table_hbm, idx_hbm, tgt_hbm, lse_hbm, out_hbm, part_hbm,
               idx_v, tgt_v, lse_v, buf_a, buf_b, acc_v, sem_a, sem_b):
    wid = lax.axis_index("s") * _NC + lax.axis_index("c")
    base = wid * _PER_W
    pltpu.sync_copy(idx_hbm.at[wid], idx_v)
    pltpu.sync_copy(tgt_hbm.at[wid], tgt_v)
    pltpu.sync_copy(lse_hbm, lse_v)
    acc_v[...] = jnp.zeros((16,), jnp.float32)

    def gather_start(j, buf, sem):
        pltpu.async_copy(table_hbm.at[idx_v.at[j]], buf, sem)

    def gather_wait(j, buf, sem):
        pltpu.make_async_copy(table_hbm.at[idx_v.at[j]], buf, sem).wait()

    def consume(j, buf):
        for b in range(_CHUNK // 16):
            rid = lax.iota(jnp.int32, 16) + (b * 16)
            tg = tgt_v[j, pl.ds(b * 16, 16)]
            ii = idx_v[j, pl.ds(b * 16, 16)]
            picked = plsc.load_gather(buf, [rid, tg])
            lseg = plsc.load_gather(lse_v, [ii])
            acc_v[...] = acc_v[...] + (lseg - picked)
        pltpu.sync_copy(buf, out_hbm.at[pl.ds(base + j * _CHUNK, _CHUNK)])

    gather_start(0, buf_a, sem_a)

    def pair_body(t, carry):
        j0 = 2 * t
        gather_wait(j0, buf_a, sem_a)
        gather_start(j0 + 1, buf_b, sem_b)
        consume(j0, buf_a)

        gather_wait(j0 + 1, buf_b, sem_b)

        @pl.when(t + 1 < _NCHUNK // 2)
        def _():
            gather_start(j0 + 2, buf_a, sem_a)

        consume(j0 + 1, buf_b)
        return carry

    lax.fori_loop(0, _NCHUNK // 2, pair_body, 0)
    pltpu.sync_copy(acc_v, part_hbm.at[wid])


def kernel(idx, targets, token_emb_table):
    idx_r = idx.reshape(_NW, _NCHUNK, _CHUNK)
    tgt_r = targets.reshape(_NW, _NCHUNK, _CHUNK)
    lse = _lse_call(token_emb_table).reshape(_V)
    logits2, partials = _sc_gather(token_emb_table, idx_r, tgt_r, lse)
    loss = _loss_call(partials)[0, 0]
    return (logits2, loss)


# speedup vs baseline: 1.5793x; 1.0553x over previous
import functools

import jax
import jax.numpy as jnp
from jax import lax
from jax.experimental import pallas as pl
from jax.experimental.pallas import tpu as pltpu
from jax.experimental.pallas import tpu_sc as plsc

_V = 1000
_N = 204800
_NC, _NS = 2, 16
_NW = _NC * _NS
_PER_W = _N // _NW
_NCG = _V // 8
_NTI = _N // 128
_TPW = _NTI // _NW
_HT = _TPW // 2


def _lse_body(t_ref, o_ref):
    t = t_ref[...]
    m = jnp.max(t, axis=1, keepdims=True)
    o_ref[...] = m + jnp.log(jnp.sum(jnp.exp(t - m), axis=1, keepdims=True))


_lse_call = pl.pallas_call(
    _lse_body,
    out_shape=jax.ShapeDtypeStruct((_V, 1), jnp.float32),
)


def _loss_body(p_ref, o_ref):
    o_ref[0, 0] = jnp.sum(p_ref[...]) * (1.0 / _N)


_loss_call = pl.pallas_call(
    _loss_body,
    out_shape=jax.ShapeDtypeStruct((1, 1), jnp.float32),
    out_specs=pl.BlockSpec(memory_space=pltpu.SMEM),
)


_mesh = plsc.VectorSubcoreMesh(core_axis_name="c", subcore_axis_name="s")


@functools.partial(
    pl.kernel,
    out_type=(
        jax.ShapeDtypeStruct((_NCG, _NTI, 8, 128), jnp.float32),
        jax.ShapeDtypeStruct((_NW, 16), jnp.float32),
    ),
    mesh=_mesh,
    compiler_params=pltpu.CompilerParams(
        use_tc_tiling_on_sc=False, needs_layout_passes=False),
    scratch_types=[
        pltpu.VMEM((_PER_W,), jnp.int32),
        pltpu.VMEM((_PER_W,), jnp.int32),
        pltpu.VMEM((_PER_W,), jnp.float32),
        pltpu.VMEM((_V,), jnp.float32),
        pltpu.VMEM((8 * _V,), jnp.float32),
        pltpu.VMEM((8 * _V,), jnp.float32),
        pltpu.VMEM((_HT, 8, 128), jnp.float32),
        pltpu.VMEM((_HT, 8, 128), jnp.float32),
        pltpu.VMEM((16,), jnp.float32),
        pltpu.SemaphoreType.DMA,
        pltpu.SemaphoreType.DMA,
        pltpu.SemaphoreType.DMA,
        pltpu.SemaphoreType.DMA,
        pltpu.SemaphoreType.DMA,
    ],
)
def _sc_main(tt_hbm, idx_hbm, tgt_hbm, lse_hbm, out_hbm, part_hbm,
             idx_v, off_v, picked_v, lse_v, tt_a, tt_b, st_a, st_b,
             acc_v, sem_ta, sem_tb, sem_sa, sem_sb, sem_p):
    wid = lax.axis_index("s") * _NC + lax.axis_index("c")
    ebase = wid * _PER_W
    tbase = wid * _TPW

    pltpu.sync_copy(idx_hbm.at[pl.ds(ebase, _PER_W)], idx_v)
    pltpu.sync_copy(tgt_hbm.at[pl.ds(ebase, _PER_W)], off_v)
    pltpu.sync_copy(lse_hbm, lse_v)
    acc_v[...] = jnp.zeros((16,), jnp.float32)

    def off_body(g, carry):
        sl = pl.ds(g * 16, 16)
        off_v[sl] = off_v[sl] * _V + idx_v[sl]
        return carry

    lax.fori_loop(0, _PER_W // 16, off_body, 0)

    for c in range(_PER_W // 128):
        pltpu.async_copy(
            tt_hbm.at[off_v.at[pl.ds(c * 128, 128)]],
            picked_v.at[pl.ds(c * 128, 128)],
            sem_p,
        )
    for c in range(_PER_W // 128):
        pltpu.make_async_copy(
            tt_hbm.at[off_v.at[pl.ds(c * 128, 128)]],
            picked_v.at[pl.ds(c * 128, 128)],
            sem_p,
        ).wait()

    def lp_body(g, carry):
        sl = pl.ds(g * 16, 16)
        lseg = plsc.load_gather(lse_v, [idx_v[sl]])
        acc_v[...] = acc_v[...] + (lseg - picked_v[sl])
        return carry

    lax.fori_loop(0, _PER_W // 16, lp_body, 0)
    pltpu.sync_copy(acc_v, part_hbm.at[wid])

    def tt_start(cg, tt, sem):
        pltpu.async_copy(tt_hbm.at[pl.ds(cg * (8 * _V), 8 * _V)], tt, sem)

    def tt_wait(cg, tt, sem):
        pltpu.make_async_copy(
            tt_hbm.at[pl.ds(cg * (8 * _V), 8 * _V)], tt, sem).wait()

    def st_start(cg, h, st, sem):
        pltpu.async_copy(st, out_hbm.at[cg, pl.ds(tbase + h * _HT, _HT)], sem)

    def st_wait(cg, h, st, sem):
        pltpu.make_async_copy(
            st, out_hbm.at[cg, pl.ds(tbase + h * _HT, _HT)], sem).wait()

    def fill_half(st, tt):
        def ip_body(ip, e0):
            for m in range(8):
                iv = idx_v[pl.ds(e0 + m * 16, 16)]
                for s in range(8):
                    st[ip, s, pl.ds(m * 16, 16)] = plsc.load_gather(
                        tt, [iv + (s * _V)])
            return e0 + 128

        return ip_body

    def do_group(cg, tt, h, st):
        body = fill_half(st, tt)
        lax.fori_loop(0, _HT, body, h * (_HT * 128))

    tt_start(0, tt_a, sem_ta)

    def pair_body(t, carry):
        c0 = 2 * t

        tt_wait(c0, tt_a, sem_ta)
        tt_start(c0 + 1, tt_b, sem_tb)

        @pl.when(t > 0)
        def _():
            st_wait(c0 - 1, 0, st_a, sem_sa)

        do_group(c0, tt_a, 0, st_a)
        st_start(c0, 0, st_a, sem_sa)

        @pl.when(t > 0)
        def _():
            st_wait(c0 - 1, 1, st_b, sem_sb)

        do_group(c0, tt_a, 1, st_b)
        st_start(c0, 1, st_b, sem_sb)

        tt_wait(c0 + 1, tt_b, sem_tb)
        tt_start(c0 + 2, tt_a, sem_ta)
        st_wait(c0, 0, st_a, sem_sa)
        do_group(c0 + 1, tt_b, 0, st_a)
        st_start(c0 + 1, 0, st_a, sem_sa)
        st_wait(c0, 1, st_b, sem_sb)
        do_group(c0 + 1, tt_b, 1, st_b)
        st_start(c0 + 1, 1, st_b, sem_sb)
        return carry

    lax.fori_loop(0, (_NCG - 1) // 2, pair_body, 0)

    cg = _NCG - 1
    tt_wait(cg, tt_a, sem_ta)
    st_wait(cg - 1, 0, st_a, sem_sa)
    do_group(cg, tt_a, 0, st_a)
    st_start(cg, 0, st_a, sem_sa)
    st_wait(cg - 1, 1, st_b, sem_sb)
    do_group(cg, tt_a, 1, st_b)
    st_start(cg, 1, st_b, sem_sb)
    st_wait(cg, 0, st_a, sem_sa)
    st_wait(cg, 1, st_b, sem_sb)


def kernel(idx, targets, token_emb_table):
    tableT_flat = token_emb_table.T.reshape(_V * _V)
    idx_f = idx.reshape(_N)
    tgt_f = targets.reshape(_N)
    lse = _lse_call(token_emb_table).reshape(_V)
    y4, partials = _sc_main(tableT_flat, idx_f, tgt_f, lse)
    logits2 = y4.transpose(0, 2, 1, 3).reshape(_V, _N).T
    loss = _loss_call(partials)[0, 0]
    return (logits2, loss)


# speedup vs baseline: 5.4279x; 3.4370x over previous
import functools

import jax
import jax.numpy as jnp
from jax import lax
from jax.experimental import pallas as pl
from jax.experimental.pallas import tpu as pltpu
from jax.experimental.pallas import tpu_sc as plsc

_V = 1000
_N = 204800
_NC, _NS = 2, 16
_NW = _NC * _NS
_PER_W = _N // _NW
_NCG = _V // 8
_NTI = _N // 128
_TPW = _NTI // _NW
_HT = _TPW // 2


def _lse_body(t_ref, o_ref):
    t = t_ref[...]
    m = jnp.max(t, axis=1, keepdims=True)
    o_ref[...] = m + jnp.log(jnp.sum(jnp.exp(t - m), axis=1, keepdims=True))


_lse_call = pl.pallas_call(
    _lse_body,
    out_shape=jax.ShapeDtypeStruct((_V, 1), jnp.float32),
)


def _loss_body(p_ref, o_ref):
    o_ref[0, 0] = jnp.sum(p_ref[...]) * (1.0 / _N)


_loss_call = pl.pallas_call(
    _loss_body,
    out_shape=jax.ShapeDtypeStruct((1, 1), jnp.float32),
    out_specs=pl.BlockSpec(memory_space=pltpu.SMEM),
)


_mesh = plsc.VectorSubcoreMesh(core_axis_name="c", subcore_axis_name="s")


@functools.partial(
    pl.kernel,
    out_type=(
        jax.ShapeDtypeStruct((_NCG, _NTI, 8, 128), jnp.float32),
        jax.ShapeDtypeStruct((_NW, 16), jnp.float32),
    ),
    mesh=_mesh,
    compiler_params=pltpu.CompilerParams(
        use_tc_tiling_on_sc=False, needs_layout_passes=False),
    scratch_types=[
        pltpu.VMEM((_PER_W,), jnp.int32),
        pltpu.VMEM((_PER_W,), jnp.int32),
        pltpu.VMEM((_PER_W,), jnp.float32),
        pltpu.VMEM((_V,), jnp.float32),
        pltpu.VMEM((8 * _V,), jnp.float32),
        pltpu.VMEM((8 * _V,), jnp.float32),
        pltpu.VMEM((_HT, 8, 128), jnp.float32),
        pltpu.VMEM((_HT, 8, 128), jnp.float32),
        pltpu.VMEM((16,), jnp.float32),
        pltpu.SemaphoreType.DMA,
        pltpu.SemaphoreType.DMA,
        pltpu.SemaphoreType.DMA,
        pltpu.SemaphoreType.DMA,
        pltpu.SemaphoreType.DMA,
    ],
)
def _sc_main(tt_hbm, idx_hbm, tgt_hbm, lse_hbm, out_hbm, part_hbm,
             idx_v, off_v, picked_v, lse_v, tt_a, tt_b, st_a, st_b,
             acc_v, sem_ta, sem_tb, sem_sa, sem_sb, sem_p):
    wid = lax.axis_index("s") * _NC + lax.axis_index("c")
    ebase = wid * _PER_W
    tbase = wid * _TPW

    pltpu.sync_copy(idx_hbm.at[pl.ds(ebase, _PER_W)], idx_v)
    pltpu.sync_copy(tgt_hbm.at[pl.ds(ebase, _PER_W)], off_v)
    pltpu.sync_copy(lse_hbm, lse_v)
    acc_v[...] = jnp.zeros((16,), jnp.float32)

    def off_body(g, carry):
        sl = pl.ds(g * 16, 16)
        off_v[sl] = off_v[sl] * _V + idx_v[sl]
        return carry

    lax.fori_loop(0, _PER_W // 16, off_body, 0)

    for c in range(_PER_W // 128):
        pltpu.async_copy(
            tt_hbm.at[off_v.at[pl.ds(c * 128, 128)]],
            picked_v.at[pl.ds(c * 128, 128)],
            sem_p,
        )
    for c in range(_PER_W // 128):
        pltpu.make_async_copy(
            tt_hbm.at[off_v.at[pl.ds(c * 128, 128)]],
            picked_v.at[pl.ds(c * 128, 128)],
            sem_p,
        ).wait()

    def lp_body(g, carry):
        sl = pl.ds(g * 16, 16)
        lseg = plsc.load_gather(lse_v, [idx_v[sl]])
        acc_v[...] = acc_v[...] + (lseg - picked_v[sl])
        return carry

    lax.fori_loop(0, _PER_W // 16, lp_body, 0)
    pltpu.sync_copy(acc_v, part_hbm.at[wid])

    def tt_start(cg, tt, sem):
        pltpu.async_copy(tt_hbm.at[pl.ds(cg * (8 * _V), 8 * _V)], tt, sem)

    def tt_wait(cg, tt, sem):
        pltpu.make_async_copy(
            tt_hbm.at[pl.ds(cg * (8 * _V), 8 * _V)], tt, sem).wait()

    def st_start(cg, h, st, sem):
        pltpu.async_copy(st, out_hbm.at[cg, pl.ds(tbase + h * _HT, _HT)], sem)

    def st_wait(cg, h, st, sem):
        pltpu.make_async_copy(
            st, out_hbm.at[cg, pl.ds(tbase + h * _HT, _HT)], sem).wait()

    def do_group(cg, tt, h, st):
        @plsc.parallel_loop(0, _HT, unroll=1)
        def _(ip):
            e0 = h * (_HT * 128) + ip * 128
            ivs = [idx_v[pl.ds(e0 + m * 16, 16)] for m in range(8)]
            for s in range(8):
                for m in range(8):
                    st[ip, s, pl.ds(m * 16, 16)] = plsc.load_gather(
                        tt, [ivs[m] + (s * _V)])

    tt_start(0, tt_a, sem_ta)

    def pair_body(t, carry):
        c0 = 2 * t

        tt_wait(c0, tt_a, sem_ta)
        tt_start(c0 + 1, tt_b, sem_tb)

        @pl.when(t > 0)
        def _():
            st_wait(c0 - 1, 0, st_a, sem_sa)

        do_group(c0, tt_a, 0, st_a)
        st_start(c0, 0, st_a, sem_sa)

        @pl.when(t > 0)
        def _():
            st_wait(c0 - 1, 1, st_b, sem_sb)

        do_group(c0, tt_a, 1, st_b)
        st_start(c0, 1, st_b, sem_sb)

        tt_wait(c0 + 1, tt_b, sem_tb)
        tt_start(c0 + 2, tt_a, sem_ta)
        st_wait(c0, 0, st_a, sem_sa)
        do_group(c0 + 1, tt_b, 0, st_a)
        st_start(c0 + 1, 0, st_a, sem_sa)
        st_wait(c0, 1, st_b, sem_sb)
        do_group(c0 + 1, tt_b, 1, st_b)
        st_start(c0 + 1, 1, st_b, sem_sb)
        return carry

    lax.fori_loop(0, (_NCG - 1) // 2, pair_body, 0)

    cg = _NCG - 1
    tt_wait(cg, tt_a, sem_ta)
    st_wait(cg - 1, 0, st_a, sem_sa)
    do_group(cg, tt_a, 0, st_a)
    st_start(cg, 0, st_a, sem_sa)
    st_wait(cg - 1, 1, st_b, sem_sb)
    do_group(cg, tt_a, 1, st_b)
    st_start(cg, 1, st_b, sem_sb)
    st_wait(cg, 0, st_a, sem_sa)
    st_wait(cg, 1, st_b, sem_sb)


def kernel(idx, targets, token_emb_table):
    tableT_flat = token_emb_table.T.reshape(_V * _V)
    idx_f = idx.reshape(_N)
    tgt_f = targets.reshape(_N)
    lse = _lse_call(token_emb_table).reshape(_V)
    y4, partials = _sc_main(tableT_flat, idx_f, tgt_f, lse)
    logits2 = y4.transpose(0, 2, 1, 3).reshape(_V, _N).T
    loss = _loss_call(partials)[0, 0]
    return (logits2, loss)
